# baseline (speedup 1.0000x reference)
import functools

import jax
import jax.numpy as jnp
from jax import lax
from jax.experimental import pallas as pl
from jax.experimental import pallas as pl_
from jax.experimental.pallas import tpu as pltpu
from jax.experimental.pallas import tpu_sc as plsc

V = 1000000
D = 64
B = 16384

NC, NS, L = 2, 16, 16
NW = NC * NS
BPW = B // NW
CHUNK = 128
NCHUNK = BPW // CHUNK
NBLK = CHUNK // L


def _body(wi_i_hbm, wi_j_hbm, wi_hbm, wj_hbm, bb_hbm, out_hbm,
          widx_i, widx_j, pidx_i, pidx_j, bidx_i, bidx_j,
          rows_i, rows_j, brow_i, brow_j, out_v, sem):
    wid = lax.axis_index("s") * NC + lax.axis_index("c")
    base = wid * BPW

    pltpu.sync_copy(wi_i_hbm.at[pl.ds(base, BPW)], widx_i)
    pltpu.sync_copy(wi_j_hbm.at[pl.ds(base, BPW)], widx_j)

    def stage(t, carry):
        s = pl.ds(t * L, L)
        wv_i = widx_i[s]
        wv_j = widx_j[s]
        pidx_i[s] = wv_i >> 1
        pidx_j[s] = wv_j >> 1
        bidx_i[s] = wv_i >> 7
        bidx_j[s] = (wv_j + V) >> 7
        return carry

    lax.fori_loop(0, BPW // L, stage, 0, unroll=False)

    iota = lax.iota(jnp.int32, L)

    def chunk(c, carry):
        c0 = c * CHUNK
        g1 = pltpu.async_copy(wi_hbm.at[pidx_i.at[pl.ds(c0, CHUNK)]], rows_i, sem)
        g2 = pltpu.async_copy(wj_hbm.at[pidx_j.at[pl.ds(c0, CHUNK)]], rows_j, sem)
        g3 = pltpu.async_copy(bb_hbm.at[bidx_i.at[pl.ds(c0, CHUNK)]], brow_i, sem)
        g4 = pltpu.async_copy(bb_hbm.at[bidx_j.at[pl.ds(c0, CHUNK)]], brow_j, sem)
        g1.wait()
        g2.wait()
        g3.wait()
        g4.wait()

        def block(b, carry2):
            g0 = c0 + b * L
            s = pl.ds(g0, L)
            lrvec = b * L + iota
            wv_i = widx_i[s]
            wv_j = widx_j[s]
            col_i = (wv_i & 1) * D
            col_j = (wv_j & 1) * D
            acc = plsc.load_gather(brow_i, [lrvec, wv_i & 127]) + \
                plsc.load_gather(brow_j, [lrvec, (wv_j + V) & 127])
            for d in range(D):
                gi = plsc.load_gather(rows_i, [lrvec, col_i + d])
                gj = plsc.load_gather(rows_j, [lrvec, col_j + d])
                acc = acc + gi * gj
            out_v[s] = acc
            return carry2

        lax.fori_loop(0, NBLK, block, 0, unroll=False)
        return carry

    lax.fori_loop(0, NCHUNK, chunk, 0, unroll=False)

    pltpu.sync_copy(out_v, out_hbm.at[pl.ds(base, BPW)])


@functools.partial(jax.jit, static_argnames=())
def kernel(word_i, word_j, wi, wj, bi, bj):
    mesh = plsc.VectorSubcoreMesh(core_axis_name="c", subcore_axis_name="s")
    k = pl.kernel(
        _body,
        out_type=jax.ShapeDtypeStruct((B,), jnp.float32),
        mesh=mesh,
        compiler_params=pltpu.CompilerParams(needs_layout_passes=False),
        scratch_types=[
            pltpu.VMEM((BPW,), jnp.int32),
            pltpu.VMEM((BPW,), jnp.int32),
            pltpu.VMEM((BPW,), jnp.int32),
            pltpu.VMEM((BPW,), jnp.int32),
            pltpu.VMEM((BPW,), jnp.int32),
            pltpu.VMEM((BPW,), jnp.int32),
            pltpu.VMEM((CHUNK, 2 * D), jnp.float32),
            pltpu.VMEM((CHUNK, 2 * D), jnp.float32),
            pltpu.VMEM((CHUNK, 2 * D), jnp.float32),
            pltpu.VMEM((CHUNK, 2 * D), jnp.float32),
            pltpu.VMEM((BPW,), jnp.float32),
            pltpu.SemaphoreType.DMA,
        ],
    )
    wi2 = wi.reshape(V // 2, 2 * D)
    wj2 = wj.reshape(V // 2, 2 * D)
    bb = jnp.concatenate([bi.reshape(V), bj.reshape(V)]).reshape(2 * V // 128, 128)
    return k(word_i.astype(jnp.int32), word_j.astype(jnp.int32), wi2, wj2, bb)
